# baseline (speedup 1.0000x reference)
import functools

import jax
import jax.numpy as jnp
import numpy as np
from jax.experimental import pallas as pl
from jax.experimental.pallas import tpu as pltpu

B = 128
L = 196
DIM = 256
NH = 8
HD = 32
NE = 64
TOP_K = 8
CAPACITY = 160
EPS = 1e-6
AUX_W = 0.01

_SCALE = np.float32(1.0 / np.sqrt(HD))
_INV196 = np.float32(1.0 / 196.0)
_INV8 = np.float32(0.125)
_INV128 = np.float32(1.0 / 128.0)
_INV64 = np.float32(1.0 / 64.0)


def _dot(a, b):
    return jnp.dot(a, b, preferred_element_type=jnp.float32)


def _halves_tree(acc):
    for s in (4, 2, 1):
        acc = acc + pltpu.roll(acc, s, axis=0)
    return acc[7:8]


def _sum64_cols(e):
    y = e.T
    acc = y[0:8]
    for c in range(1, 8):
        acc = acc + y[8 * c:8 * c + 8]
    return _halves_tree(acc).T


def _sum196_cols(e):
    y = e.T
    acc = y[0:8]
    for c in range(1, 24):
        acc = acc + y[8 * c:8 * c + 8]
    acc = acc + jnp.concatenate([y[192:196], jnp.zeros((4, 196), jnp.float32)], axis=0)
    return _halves_tree(acc).T


def _sum196_rows(x):
    c = x.shape[1]
    y = x.T
    tail = jnp.concatenate([y[:, 128:196], jnp.zeros((c, 60), jnp.float32)], axis=1)
    f = y[:, 0:128] + tail
    return jnp.sum(f, axis=1, keepdims=True).T


def _k2_kernel(eq_ref, wk2t_ref, bk2_ref, o_ref):
    eq16 = eq_ref[...].astype(jnp.bfloat16)
    w16 = wk2t_ref[...].astype(jnp.bfloat16)
    o_ref[...] = _dot(eq16, w16) + bk2_ref[...]


def _main_kernel(x_ref, wqt_ref, wkt_ref, wvt_ref, wot_ref, wq2t_ref,
                 bq_ref, bk_ref, bv_ref, bo_ref, bq2_ref, kp2_ref, ses_ref):
    xb = x_ref[0]
    x16 = xb.astype(jnp.bfloat16)
    q = _dot(x16, wqt_ref[...].astype(jnp.bfloat16)) + bq_ref[...]
    k = _dot(x16, wkt_ref[...].astype(jnp.bfloat16)) + bk_ref[...]
    v = _dot(x16, wvt_ref[...].astype(jnp.bfloat16)) + bv_ref[...]
    q16 = q.astype(jnp.bfloat16)
    k16 = k.astype(jnp.bfloat16)
    v16 = v.astype(jnp.bfloat16)

    outs = []
    for h in range(NH):
        qh = q16[:, HD * h:HD * h + HD]
        kh = k16[:, HD * h:HD * h + HD]
        vh = v16[:, HD * h:HD * h + HD]
        lg = _dot(qh, kh.T) * _SCALE
        m = jnp.max(lg, axis=1, keepdims=True)
        e = jnp.exp(lg - m)
        s = _sum196_cols(e)
        attn = (e / s).astype(jnp.bfloat16)
        outs.append(_dot(attn, vh).astype(jnp.bfloat16))
    o16 = jnp.concatenate(outs, axis=1)

    ctx = _dot(o16, wot_ref[...].astype(jnp.bfloat16)) + bo_ref[...]
    q2 = _dot(ctx.astype(jnp.bfloat16),
              wq2t_ref[...].astype(jnp.bfloat16)) + bq2_ref[...]
    q216 = q2.astype(jnp.bfloat16)
    kp2_16 = kp2_ref[...].astype(jnp.bfloat16)

    hacc = jnp.zeros((L, NE), jnp.float32)
    for h in range(NH):
        q2h = q216[:, HD * h:HD * h + HD]
        k2h = kp2_16[:, HD * h:HD * h + HD]
        lg2 = _dot(q2h, k2h.T) * _SCALE
        m2 = jnp.max(lg2, axis=1, keepdims=True)
        e2 = jnp.exp(lg2 - m2)
        s2 = _sum64_cols(e2)
        hacc = hacc + e2 / s2
    hm = hacc * _INV8
    ses_ref[...] = _sum196_rows(hm).reshape(1, 1, NE)


def _gate_kernel(ses_ref, gate_ref, aux_ref):
    t = ses_ref[...] * _INV196
    m = jnp.max(t, axis=1, keepdims=True)
    e = jnp.exp(t - m)
    s = _sum64_cols(e)
    rgs = e / s

    iota = jax.lax.broadcasted_iota(jnp.int32, (B, NE), 1)
    work = rgs
    mask = jnp.zeros((B, NE), jnp.float32)
    for _ in range(TOP_K):
        mx = jnp.max(work, axis=1, keepdims=True)
        cand = jnp.where(work == mx, iota, NE)
        idx = jnp.min(cand, axis=1, keepdims=True)
        sel = iota == idx
        mask = mask + sel.astype(jnp.float32)
        work = jnp.where(sel, -jnp.inf, work)

    masked = rgs * mask
    den = jnp.sum(masked, axis=0, keepdims=True) + EPS
    gate_ref[...] = masked / den * np.float32(CAPACITY)

    importance = jnp.sum(rgs, axis=0, keepdims=True) * _INV128
    load = jnp.sum(mask, axis=0, keepdims=True) * _INV128
    aux = jnp.sum(importance * load, axis=1, keepdims=True) * _INV64
    aux_ref[...] = aux * np.float32(AUX_W) * np.float32(NE * NE)


@jax.jit
def kernel(x, sa_w_in, sa_b_in, sa_w_out, sa_b_out, ca_w_in, ca_b_in,
           ca_w_out, ca_b_out, expert_queries):
    xs = x.reshape(B, L, DIM)
    wqt = sa_w_in[0:DIM].T
    wkt = sa_w_in[DIM:2 * DIM].T
    wvt = sa_w_in[2 * DIM:3 * DIM].T
    wot = sa_w_out.T
    wq2t = ca_w_in[0:DIM].T
    wk2t = ca_w_in[DIM:2 * DIM].T
    bq = sa_b_in[0:DIM].reshape(1, DIM)
    bk = sa_b_in[DIM:2 * DIM].reshape(1, DIM)
    bv = sa_b_in[2 * DIM:3 * DIM].reshape(1, DIM)
    bo = sa_b_out.reshape(1, DIM)
    bq2 = ca_b_in[0:DIM].reshape(1, DIM)
    bk2 = ca_b_in[DIM:2 * DIM].reshape(1, DIM)

    kp2 = pl.pallas_call(
        _k2_kernel,
        out_shape=jax.ShapeDtypeStruct((NE, DIM), jnp.float32),
    )(expert_queries, wk2t, bk2)

    full = lambda *shape: pl.BlockSpec(shape, lambda i: tuple(0 for _ in shape))
    ses = pl.pallas_call(
        _main_kernel,
        grid=(B,),
        in_specs=[
            pl.BlockSpec((1, L, DIM), lambda i: (i, 0, 0)),
            full(DIM, DIM), full(DIM, DIM), full(DIM, DIM), full(DIM, DIM),
            full(DIM, DIM),
            full(1, DIM), full(1, DIM), full(1, DIM), full(1, DIM),
            full(1, DIM), full(NE, DIM),
        ],
        out_specs=pl.BlockSpec((1, 1, NE), lambda i: (i, 0, 0)),
        out_shape=jax.ShapeDtypeStruct((B, 1, NE), jnp.float32),
    )(xs, wqt, wkt, wvt, wot, wq2t, bq, bk, bv, bo, bq2, kp2)

    gate, aux = pl.pallas_call(
        _gate_kernel,
        out_shape=(jax.ShapeDtypeStruct((B, NE), jnp.float32),
                   jax.ShapeDtypeStruct((1, 1), jnp.float32)),
    )(ses.reshape(B, NE))

    return (gate, aux.reshape(()))
